# baseline (speedup 1.0000x reference)
import functools

import jax
import jax.numpy as jnp
from jax import lax
from jax.experimental import pallas as pl
from jax.experimental.pallas import tpu as pltpu
from jax.experimental.pallas import tpu_sc as plsc

B = 160000
D = 256
NC = 2
NS = 16
NW = NC * NS
BPW = B // NW
C = 240
CHUNKS = [C] * (BPW // C) + ([BPW % C] if BPW % C else [])
OFFS = [C * i for i in range(len(CHUNKS))]
N = len(CHUNKS)


def _gather_body(table_hbm, idx_hbm, out_hbm, idx_v, rows0, rows1,
                 gsem0, gsem1, wsem0, wsem1):
    wid = lax.axis_index("s") * NC + lax.axis_index("c")
    base = wid * BPW
    rows = (rows0, rows1)
    gsem = (gsem0, gsem1)
    wsem = (wsem0, wsem1)

    pltpu.sync_copy(idx_hbm.at[pl.ds(base, BPW)], idx_v)

    def gather_desc(g):
        s, n = g % 2, CHUNKS[g]
        dst = rows[s] if n == C else rows[s].at[pl.ds(0, n)]
        return pltpu.make_async_copy(
            table_hbm.at[idx_v.at[pl.ds(OFFS[g], n)]], dst, gsem[s])

    def write_desc(g):
        s, n = g % 2, CHUNKS[g]
        src = rows[s] if n == C else rows[s].at[pl.ds(0, n)]
        return pltpu.make_async_copy(
            src, out_hbm.at[pl.ds(base + OFFS[g], n)], wsem[s])

    gather_desc(0).start()
    for g in range(N):
        if g + 1 < N:
            if g >= 1:
                write_desc(g - 1).wait()
            gather_desc(g + 1).start()
        gather_desc(g).wait()
        write_desc(g).start()
    write_desc(N - 2).wait()
    write_desc(N - 1).wait()


_sc_gather = functools.partial(
    pl.kernel,
    out_type=jax.ShapeDtypeStruct((B, D), jnp.float32),
    mesh=plsc.VectorSubcoreMesh(core_axis_name="c", subcore_axis_name="s"),
    scratch_types=[
        pltpu.VMEM((BPW,), jnp.int32),
        pltpu.VMEM((C, D), jnp.float32),
        pltpu.VMEM((C, D), jnp.float32),
        pltpu.SemaphoreType.DMA,
        pltpu.SemaphoreType.DMA,
        pltpu.SemaphoreType.DMA,
        pltpu.SemaphoreType.DMA,
    ],
)(_gather_body)


def kernel(features, neighbor_indices):
    table = features[0]
    idx = neighbor_indices.reshape(B)
    out = _sc_gather(table, idx)
    return out[None]


# speedup vs baseline: 1.0341x; 1.0341x over previous
import functools

import jax
import jax.numpy as jnp
from jax import lax
from jax.experimental import pallas as pl
from jax.experimental.pallas import tpu as pltpu
from jax.experimental.pallas import tpu_sc as plsc

B = 160000
D = 256
NC = 2
NS = 16
NW = NC * NS
BPW = B // NW
C = 160
NFULL = BPW // C
TAIL = BPW - NFULL * C
NBUF = 3


def _gather_body(table_hbm, idx_hbm, out_hbm, idx_v, rows0, rows1, rows2,
                 gsem0, gsem1, gsem2, wsem0, wsem1, wsem2):
    wid = lax.axis_index("s") * NC + lax.axis_index("c")
    base = wid * BPW
    rows = (rows0, rows1, rows2)
    gsem = (gsem0, gsem1, gsem2)
    wsem = (wsem0, wsem1, wsem2)

    pltpu.sync_copy(idx_hbm.at[pl.ds(base, BPW)], idx_v)

    def gather_desc(s, g, n=C):
        dst = rows[s] if n == C else rows[s].at[pl.ds(0, n)]
        return pltpu.make_async_copy(
            table_hbm.at[idx_v.at[pl.ds(g * C, n)]], dst, gsem[s])

    def write_desc(s, g, n=C):
        src = rows[s] if n == C else rows[s].at[pl.ds(0, n)]
        return pltpu.make_async_copy(
            src, out_hbm.at[pl.ds(base + g * C, n)], wsem[s])

    gather_desc(0, 0).start()
    gather_desc(1, 1).start()

    def outer(k, carry):
        for b in range(NBUF):
            g = NBUF * k + b

            @pl.when(g + 2 < NFULL)
            def _():
                @pl.when(g > 0)
                def _():
                    write_desc((b + 2) % NBUF, g - 1).wait()
                gather_desc((b + 2) % NBUF, g + 2).start()

            @pl.when(g < NFULL)
            def _():
                gather_desc(b, g).wait()
                write_desc(b, g).start()
        return carry

    lax.fori_loop(0, (NFULL + NBUF - 1) // NBUF, outer, 0)

    ts = (NFULL - 3) % NBUF
    write_desc(ts, NFULL - 3).wait()
    gather_desc(ts, NFULL, TAIL).start()
    gather_desc(ts, NFULL, TAIL).wait()
    write_desc(ts, NFULL, TAIL).start()
    write_desc((NFULL - 2) % NBUF, NFULL - 2).wait()
    write_desc((NFULL - 1) % NBUF, NFULL - 1).wait()
    write_desc(ts, NFULL, TAIL).wait()


_sc_gather = functools.partial(
    pl.kernel,
    out_type=jax.ShapeDtypeStruct((B, D), jnp.float32),
    mesh=plsc.VectorSubcoreMesh(core_axis_name="c", subcore_axis_name="s"),
    scratch_types=[
        pltpu.VMEM((BPW,), jnp.int32),
        pltpu.VMEM((C, D), jnp.float32),
        pltpu.VMEM((C, D), jnp.float32),
        pltpu.VMEM((C, D), jnp.float32),
        pltpu.SemaphoreType.DMA,
        pltpu.SemaphoreType.DMA,
        pltpu.SemaphoreType.DMA,
        pltpu.SemaphoreType.DMA,
        pltpu.SemaphoreType.DMA,
        pltpu.SemaphoreType.DMA,
    ],
)(_gather_body)


def kernel(features, neighbor_indices):
    table = features[0]
    idx = neighbor_indices.reshape(B)
    out = _sc_gather(table, idx)
    return out[None]
